# baseline (speedup 1.0000x reference)
import jax
import jax.numpy as jnp
from jax import lax
from jax.experimental import pallas as pl
from jax.experimental.pallas import tpu as pltpu
from jax.experimental.pallas import tpu_sc as plsc

_PROJ = 64
_DEPTH = 20
_L = 16
_PAD = 32
_TILE = 128
_SPLIT = 8
_MAXFETCH = 1 + (_DEPTH - _SPLIT)


def _bf16_trunc(v):
    b = plsc.bitcast(v, jnp.uint32)
    r = b + jnp.uint32(0x7FFF) + ((b >> jnp.uint32(16)) & jnp.uint32(1))
    return plsc.bitcast(r & jnp.uint32(0xFFFF0000), jnp.float32)


def _sc_body(x_idx_hbm, dir_hbm, label_hbm, e1t_hbm, e2t_hbm,
             out_hbm, tgt_hbm,
             xs, dsm, lv, blocks, outv, tgtv,
             sem0, sem1, sem2):
    cid = lax.axis_index("c")
    sid = lax.axis_index("s")

    def half(lo, hi):
        cp_x = pltpu.async_copy(x_idx_hbm, xs.at[pl.ds(0, 1)], sem0)
        cp_d = pltpu.async_copy(dir_hbm, dsm.at[pl.ds(0, _DEPTH)], sem1)
        cp_l = pltpu.async_copy(label_hbm.at[0], lv.at[pl.ds(0, _DEPTH)], sem2)
        cp_x.wait()
        cp_d.wait()

        x0 = xs[pl.ds(0, _L)][0]
        d_off, d_col = {}, {}
        for c in range(_PAD // _L):
            if not any(lo <= i < hi for i in range(c * _L, (c + 1) * _L)):
                continue
            dvec = dsm[pl.ds(c * _L, _L)]
            ovec = (dvec >> jnp.int32(7)) << jnp.int32(7)
            cvec = dvec & jnp.int32(_TILE - 1)
            for j in range(_L):
                i = c * _L + j
                if lo <= i < hi:
                    d_off[i] = ovec[j]
                    d_col[i] = cvec[j]

        fetches = [(e1t_hbm, (x0 >> jnp.int32(7)) << jnp.int32(7),
                    x0 & jnp.int32(_TILE - 1))]
        fetches += [(e2t_hbm, d_off[i], d_col[i]) for i in range(lo, hi)]

        cps = [
            pltpu.async_copy(src.at[:, pl.ds(pl.multiple_of(off, _TILE),
                                             _TILE)],
                             blocks.at[slot], sem1)
            for slot, (src, off, _) in enumerate(fetches)
        ]
        for cp in cps:
            cp.wait()

        rows = [lax.iota(jnp.int32, _L) + jnp.int32(k * _L)
                for k in range(_PROJ // _L)]
        lanes = lax.iota(jnp.int32, _L)

        def extract_column(slot, col):
            cvec = jnp.full((_L,), col, jnp.int32)
            svec = jnp.full((_L,), slot, jnp.int32)
            return [_bf16_trunc(plsc.load_gather(blocks, [svec, rows[k], cvec]))
                    for k in range(_PROJ // _L)]

        proj = extract_column(0, fetches[0][2])
        acc = [jnp.zeros((_L,), jnp.float32) for _ in range(_PAD // _L)]
        for slot, i in enumerate(range(lo, hi), start=1):
            chunks = extract_column(slot, fetches[slot][2])
            d = proj[0] * chunks[0]
            for k in range(1, _PROJ // _L):
                d = d + proj[k] * chunks[k]
            s = jnp.sum(d)
            acc[i // _L] = jnp.where(lanes == (i % _L), s, acc[i // _L])

        cp_l.wait()
        for c in range(_PAD // _L):
            if not any(lo <= i < hi for i in range(c * _L, (c + 1) * _L)):
                continue
            out = 1.0 / (1.0 + jnp.exp(-acc[c]))
            mask = jnp.where(out >= 0.5, 1, 0)
            lab = lv[pl.ds(c * _L, _L)]
            tgt = jnp.where(mask == lab, 1, 0)
            outv[pl.ds(c * _L, _L)] = out
            tgtv[pl.ds(c * _L, _L)] = tgt

        n = hi - lo
        cp_o = pltpu.async_copy(outv.at[pl.ds(lo, n)],
                                out_hbm.at[0].at[pl.ds(lo, n)], sem0)
        cp_t = pltpu.async_copy(tgtv.at[pl.ds(lo, n)],
                                tgt_hbm.at[0].at[pl.ds(lo, n)], sem2)
        cp_o.wait()
        cp_t.wait()

    @pl.when(jnp.logical_and(cid == 0, sid == 0))
    def _():
        half(0, _SPLIT)

    @pl.when(jnp.logical_and(cid == 1, sid == 0))
    def _():
        half(_SPLIT, _DEPTH)


def _compiler_params():
    return pltpu.CompilerParams(needs_layout_passes=False,
                                disable_bounds_checks=True,
                                disable_semaphore_checks=True)


def _run(x_idx, dir_path, label, emb1_t, emb2_t):
    call = pl.kernel(
        _sc_body,
        compiler_params=_compiler_params(),
        out_type=(jax.ShapeDtypeStruct((1, _DEPTH), jnp.float32),
                  jax.ShapeDtypeStruct((1, _DEPTH), jnp.int32)),
        mesh=plsc.VectorSubcoreMesh(core_axis_name="c", subcore_axis_name="s",
                                    num_cores=2, num_subcores=16),
        scratch_types=[
            pltpu.VMEM((_L,), jnp.int32),
            pltpu.VMEM((_PAD,), jnp.int32),
            pltpu.VMEM((_PAD,), jnp.int32),
            pltpu.VMEM((_MAXFETCH, _PROJ, _TILE), jnp.float32),
            pltpu.VMEM((_PAD,), jnp.float32),
            pltpu.VMEM((_PAD,), jnp.int32),
            pltpu.SemaphoreType.DMA,
            pltpu.SemaphoreType.DMA,
            pltpu.SemaphoreType.DMA,
        ],
    )
    return call(x_idx, dir_path, label, emb1_t, emb2_t)


def kernel(x_idx, dir_path, label, emb1, emb2):
    out, tgt = _run(x_idx.astype(jnp.int32), dir_path.astype(jnp.int32),
                    label.astype(jnp.int32), emb1.T, emb2.T)
    return (out, tgt.astype(label.dtype))


# speedup vs baseline: 1.0037x; 1.0037x over previous
import jax
import jax.numpy as jnp
from jax import lax
from jax.experimental import pallas as pl
from jax.experimental.pallas import tpu as pltpu
from jax.experimental.pallas import tpu_sc as plsc

_PROJ = 64
_DEPTH = 20
_L = 16
_PAD = 32
_TILE = 128
_SPLIT = 8
_MAXFETCH = 1 + (_DEPTH - _SPLIT)


def _bf16_trunc(v):
    b = plsc.bitcast(v, jnp.uint32)
    r = b + jnp.uint32(0x7FFF) + ((b >> jnp.uint32(16)) & jnp.uint32(1))
    return plsc.bitcast(r & jnp.uint32(0xFFFF0000), jnp.float32)


def _sc_body(x_idx_hbm, dir_hbm, label_hbm, e1t_hbm, e2t_hbm,
             out_hbm, tgt_hbm,
             xs, dsm, offs_v, cols_v, dots_v, lv, blocks, outv, tgtv,
             sem0, sem1, sem2):
    cid = lax.axis_index("c")
    sid = lax.axis_index("s")

    lanes = lax.iota(jnp.int32, _L)
    rows = [lanes + jnp.int32(k * _L) for k in range(_PROJ // _L)]

    def extract_column(svec, cvec):
        return [_bf16_trunc(plsc.load_gather(blocks, [svec, rows[k], cvec]))
                for k in range(_PROJ // _L)]

    @pl.when(sid == 0)
    def _():
        lo = jnp.where(cid == 0, 0, _SPLIT)
        n = jnp.where(cid == 0, _SPLIT, _DEPTH - _SPLIT)

        cp_x = pltpu.async_copy(x_idx_hbm, xs.at[pl.ds(0, 1)], sem0)
        cp_d = pltpu.async_copy(dir_hbm, dsm.at[pl.ds(0, _DEPTH)], sem1)
        cp_l = pltpu.async_copy(label_hbm.at[0], lv.at[pl.ds(0, _DEPTH)], sem2)
        cp_x.wait()

        x0 = xs[pl.ds(0, _L)][0]
        pltpu.async_copy(
            e1t_hbm.at[:, pl.ds(pl.multiple_of((x0 >> jnp.int32(7))
                                               << jnp.int32(7), _TILE),
                                _TILE)],
            blocks.at[0], sem1)

        cp_d.wait()
        for c in range(_PAD // _L):
            dvec = dsm[pl.ds(c * _L, _L)]
            offs_v[pl.ds(c * _L, _L)] = (dvec >> jnp.int32(7)) << jnp.int32(7)
            cols_v[pl.ds(c * _L, _L)] = dvec & jnp.int32(_TILE - 1)
            dots_v[pl.ds(c * _L, _L)] = jnp.zeros((_L,), jnp.float32)

        @pl.loop(0, n)
        def _fire(i):
            off = plsc.load_gather(offs_v, [jnp.full((_L,), lo + i,
                                                     jnp.int32)])[0]
            pltpu.async_copy(
                e2t_hbm.at[:, pl.ds(pl.multiple_of(off, _TILE), _TILE)],
                blocks.at[i + 1], sem1)

        @pl.loop(0, n + 1)
        def _drain(i):
            pltpu.make_async_copy(e2t_hbm.at[:, pl.ds(0, _TILE)],
                                  blocks.at[0], sem1).wait()

        proj = extract_column(jnp.full((_L,), 0, jnp.int32),
                              jnp.full((_L,), x0 & jnp.int32(_TILE - 1),
                                       jnp.int32))

        @pl.loop(0, n)
        def _node(i):
            j = lo + i
            col = plsc.load_gather(cols_v, [jnp.full((_L,), j, jnp.int32)])[0]
            chunks = extract_column(jnp.full((_L,), i + 1, jnp.int32),
                                    jnp.full((_L,), col, jnp.int32))
            d = proj[0] * chunks[0]
            for k in range(1, _PROJ // _L):
                d = d + proj[k] * chunks[k]
            s = jnp.sum(d)
            plsc.store_scatter(dots_v, [jnp.full((_L,), j, jnp.int32)],
                               jnp.full((_L,), s, jnp.float32),
                               mask=(lanes == 0))

        cp_l.wait()
        for c in range(_PAD // _L):
            out = 1.0 / (1.0 + jnp.exp(-dots_v[pl.ds(c * _L, _L)]))
            mask = jnp.where(out >= 0.5, 1, 0)
            lab = lv[pl.ds(c * _L, _L)]
            tgt = jnp.where(mask == lab, 1, 0)
            outv[pl.ds(c * _L, _L)] = out
            tgtv[pl.ds(c * _L, _L)] = tgt

        @pl.when(cid == 0)
        def _():
            cp_o = pltpu.async_copy(outv.at[pl.ds(0, _SPLIT)],
                                    out_hbm.at[0].at[pl.ds(0, _SPLIT)], sem0)
            cp_t = pltpu.async_copy(tgtv.at[pl.ds(0, _SPLIT)],
                                    tgt_hbm.at[0].at[pl.ds(0, _SPLIT)], sem2)
            cp_o.wait()
            cp_t.wait()

        @pl.when(cid == 1)
        def _():
            cp_o = pltpu.async_copy(
                outv.at[pl.ds(_SPLIT, _DEPTH - _SPLIT)],
                out_hbm.at[0].at[pl.ds(_SPLIT, _DEPTH - _SPLIT)], sem0)
            cp_t = pltpu.async_copy(
                tgtv.at[pl.ds(_SPLIT, _DEPTH - _SPLIT)],
                tgt_hbm.at[0].at[pl.ds(_SPLIT, _DEPTH - _SPLIT)], sem2)
            cp_o.wait()
            cp_t.wait()


def _compiler_params():
    return pltpu.CompilerParams(needs_layout_passes=False)


def _run(x_idx, dir_path, label, emb1_t, emb2_t):
    call = pl.kernel(
        _sc_body,
        compiler_params=_compiler_params(),
        out_type=(jax.ShapeDtypeStruct((1, _DEPTH), jnp.float32),
                  jax.ShapeDtypeStruct((1, _DEPTH), jnp.int32)),
        mesh=plsc.VectorSubcoreMesh(core_axis_name="c", subcore_axis_name="s",
                                    num_cores=2, num_subcores=16),
        scratch_types=[
            pltpu.VMEM((_L,), jnp.int32),
            pltpu.VMEM((_PAD,), jnp.int32),
            pltpu.VMEM((_PAD,), jnp.int32),
            pltpu.VMEM((_PAD,), jnp.int32),
            pltpu.VMEM((_PAD,), jnp.float32),
            pltpu.VMEM((_PAD,), jnp.int32),
            pltpu.VMEM((_MAXFETCH, _PROJ, _TILE), jnp.float32),
            pltpu.VMEM((_PAD,), jnp.float32),
            pltpu.VMEM((_PAD,), jnp.int32),
            pltpu.SemaphoreType.DMA,
            pltpu.SemaphoreType.DMA,
            pltpu.SemaphoreType.DMA,
        ],
    )
    return call(x_idx, dir_path, label, emb1_t, emb2_t)


def kernel(x_idx, dir_path, label, emb1, emb2):
    out, tgt = _run(x_idx.astype(jnp.int32), dir_path.astype(jnp.int32),
                    label.astype(jnp.int32), emb1.T, emb2.T)
    return (out, tgt.astype(label.dtype))
